# baseline (speedup 1.0000x reference)
import jax
import jax.numpy as jnp
from jax.experimental import pallas as pl
from jax.experimental.pallas import tpu as pltpu
from jax.experimental.pallas import tpu_sc as plsc

_WIN = 128
_BLK = 512


def _sc_gather(idx, W, N, D):
    mesh = plsc.VectorSubcoreMesh(core_axis_name="core",
                                  subcore_axis_name="subcore")

    @pl.kernel(out_type=jax.ShapeDtypeStruct((N, D), W.dtype), mesh=mesh,
               compiler_params=pltpu.CompilerParams(use_tc_tiling_on_sc=False),
               scratch_types=[pltpu.SemaphoreType.DMA])
    def gather_kernel(w_hbm, i_hbm, o_hbm, sem):
        def body(i_vmem, o_vmem):
            copies = [
                pltpu.async_copy(
                    w_hbm.at[i_vmem.at[0, pl.ds(j * _WIN, _WIN)]],
                    o_vmem.at[pl.ds(j * _WIN, _WIN)],
                    sem,
                )
                for j in range(_BLK // _WIN)
            ]
            for c in copies:
                c.wait()

        pltpu.emit_pipeline(
            body,
            grid=(N // _BLK,),
            in_specs=[pl.BlockSpec((1, _BLK), index_map=lambda i: (0, i))],
            out_specs=[pl.BlockSpec((_BLK, D), index_map=lambda i: (i, 0))],
            core_axis_name=("core", "subcore"),
            dimension_semantics=(pltpu.PARALLEL,),
        )(i_hbm, o_hbm)

    return gather_kernel(W, idx)


def _tc_transpose(mid2d, H, B, D):
    def body(in_ref, out_ref):
        blk = in_ref[...]
        out_ref[0] = jnp.concatenate([blk[:, :D].T, blk[:, D:].T], axis=1)

    return pl.pallas_call(
        body,
        grid=(H,),
        in_specs=[pl.BlockSpec((B // 2, 128), lambda g: (g, 0))],
        out_specs=pl.BlockSpec((1, D, B), lambda g: (g, 0, 0)),
        out_shape=jax.ShapeDtypeStruct((H, D, B), jnp.float32),
    )(mid2d)


def kernel(x, W):
    B, H = x.shape
    V, D = W.shape
    N = B * H

    xt = jnp.transpose(x)
    xperm = jnp.transpose(xt.reshape(H, 2, B // 2), (0, 2, 1))
    idx = xperm.reshape(1, N)

    mid = _sc_gather(idx, W, N, D)
    mid2d = mid.reshape(N * D // 128, 128)
    out3 = _tc_transpose(mid2d, H, B, D)
    return jnp.transpose(out3, (2, 0, 1))
